# Initial kernel scaffold; baseline (speedup 1.0000x reference)
import jax
import jax.numpy as jnp
from jax.experimental import pallas as pl


def kernel(x, edge_index, edge_attr, batch, params):
    raise NotImplementedError("write your pallas kernel here")


# speedup vs baseline: 6.5406x; 6.5406x over previous
import functools

import jax
import jax.numpy as jnp
import numpy as np
from jax import lax
from jax.experimental import pallas as pl
from jax.experimental.pallas import tpu as pltpu
from jax.experimental.pallas import tpu_sc as plsc

N = 10000
E = 160000
F = 128
FE = 16
H = 128
C = 16
G = 64

NC = 2
NS = 16
NW = NC * NS
CH = 128
EP = NW * CH * 40
PER_W = EP // NW
N_CH = PER_W // CH

DV = 128
HALF = 5056
NR2 = 5064
NR = 2 * HALF

BE = 2048
NBLK = EP // BE

_SC_MESH = dict(core_axis_name="c", subcore_axis_name="s",
                num_cores=NC, num_subcores=NS)


def _sc_gather2(tabA, tabB, idxA, idxB, tok):
    DA = tabA.shape[1]
    DB = tabB.shape[1]

    def body(tabA_hbm, tabB_hbm, idxA_hbm, idxB_hbm, tok_hbm,
             outA_hbm, outB_hbm, idx_v, rowsA_v, rowsB_v, sem):
        del tok_hbm
        wid = lax.axis_index("s") * NC + lax.axis_index("c")
        base = wid * PER_W

        def step(j, carry):
            off = base + j * CH
            pltpu.sync_copy(idxA_hbm.at[pl.ds(off, CH)], idx_v)
            pltpu.async_copy(tabA_hbm.at[idx_v], rowsA_v, sem).wait()
            pltpu.sync_copy(rowsA_v, outA_hbm.at[pl.ds(off, CH)])
            pltpu.sync_copy(idxB_hbm.at[pl.ds(off, CH)], idx_v)
            pltpu.async_copy(tabB_hbm.at[idx_v], rowsB_v, sem).wait()
            pltpu.sync_copy(rowsB_v, outB_hbm.at[pl.ds(off, CH)])
            return carry

        lax.fori_loop(0, N_CH, step, 0)

    fn = pl.kernel(
        body,
        out_type=[jax.ShapeDtypeStruct((EP, DA), jnp.float32),
                  jax.ShapeDtypeStruct((EP, DB), jnp.float32)],
        mesh=plsc.VectorSubcoreMesh(**_SC_MESH),
        scratch_types=[
            pltpu.VMEM((CH,), jnp.int32),
            pltpu.VMEM((CH, DA), jnp.float32),
            pltpu.VMEM((CH, DB), jnp.float32),
            pltpu.SemaphoreType.DMA,
        ],
    )
    return fn(tabA, tabB, idxA, idxB, tok)


def _sc_scatter_add2(valsA, valsB, idx):
    def body(valA_hbm, valB_hbm, idx_hbm, outA_hbm, outB_hbm,
             idx_v, vals_v, zb_v, accA_sh, accB_sh):
        cid = lax.axis_index("c")
        sid = lax.axis_index("s")
        wid = sid * NC + cid
        base = jnp.where(sid < 8, sid * 320, 2560 + (sid - 8) * 312)

        for i in range(8):
            for j in range(DV // 16):
                zb_v[i, pl.ds(j * 16, 16)] = jnp.zeros((16,), jnp.float32)

        def zstep(r, carry):
            pltpu.sync_copy(zb_v, accA_sh.at[pl.ds(base + r * 8, 8)])
            pltpu.sync_copy(zb_v, accB_sh.at[pl.ds(base + r * 8, 8)])
            return carry

        lax.fori_loop(0, 39, zstep, 0)

        @pl.when(sid < 8)
        def _():
            pltpu.sync_copy(zb_v, accA_sh.at[pl.ds(base + 312, 8)])
            pltpu.sync_copy(zb_v, accB_sh.at[pl.ds(base + 312, 8)])

        @pl.when(sid == 15)
        def _():
            pltpu.sync_copy(zb_v, accA_sh.at[pl.ds(HALF, 8)])
            pltpu.sync_copy(zb_v, accB_sh.at[pl.ds(HALF, 8)])

        plsc.subcore_barrier()

        def do_chunks(w):
            def step(j, carry):
                off = w * PER_W + j * CH
                pltpu.sync_copy(idx_hbm.at[pl.ds(off, CH)], idx_v)
                for k in range(CH // 16):
                    v = idx_v[pl.ds(k * 16, 16)] - cid * HALF
                    v = jnp.where((v >= 0) & (v < HALF), v, HALF)
                    idx_v[pl.ds(k * 16, 16)] = v
                pltpu.sync_copy(valA_hbm.at[pl.ds(off, CH)], vals_v)
                pltpu.sync_copy(vals_v, accA_sh.at[idx_v], add=True)
                pltpu.sync_copy(valB_hbm.at[pl.ds(off, CH)], vals_v)
                pltpu.sync_copy(vals_v, accB_sh.at[idx_v], add=True)
                return carry

            lax.fori_loop(0, N_CH, step, 0)

        do_chunks(sid * NC + (1 - cid))
        do_chunks(wid)
        plsc.subcore_barrier()

        pltpu.sync_copy(accA_sh.at[pl.ds(base, 312)],
                        outA_hbm.at[pl.ds(cid * HALF + base, 312)])
        pltpu.sync_copy(accB_sh.at[pl.ds(base, 312)],
                        outB_hbm.at[pl.ds(cid * HALF + base, 312)])

        @pl.when(sid < 8)
        def _():
            pltpu.sync_copy(accA_sh.at[pl.ds(base + 312, 8)],
                            outA_hbm.at[pl.ds(cid * HALF + base + 312, 8)])
            pltpu.sync_copy(accB_sh.at[pl.ds(base + 312, 8)],
                            outB_hbm.at[pl.ds(cid * HALF + base + 312, 8)])

    fn = pl.kernel(
        body,
        out_type=[jax.ShapeDtypeStruct((NR, DV), jnp.float32),
                  jax.ShapeDtypeStruct((NR, DV), jnp.float32)],
        mesh=plsc.VectorSubcoreMesh(**_SC_MESH),
        scratch_types=[
            pltpu.VMEM((CH,), jnp.int32),
            pltpu.VMEM((CH, DV), jnp.float32),
            pltpu.VMEM((8, DV), jnp.float32),
            pltpu.VMEM_SHARED((NR2, DV), jnp.float32),
            pltpu.VMEM_SHARED((NR2, DV), jnp.float32),
        ],
    )
    return fn(valsA, valsB, idx)


def _bn(x):
    mu = jnp.mean(x, axis=0, keepdims=True)
    var = jnp.mean((x - mu) ** 2, axis=0, keepdims=True)
    return (x - mu) * lax.rsqrt(var + 1e-5) + 1e-4


def _leaky(x):
    return jnp.where(x > 0, x, 0.2 * x)


def _dot(a, b):
    return jnp.dot(a, b, preferred_element_type=jnp.float32)


def _tc_call(body, out_shapes, *args):
    return pl.pallas_call(
        body,
        out_shape=out_shapes,
    )(*args)


def _node_egat(h, Wn, We1, We2, a1, a2, prebn):
    def body(h_ref, Wn_ref, We1_ref, We2_ref, a1_ref, a2_ref,
             src_ref, dst_ref, xw_ref):
        hh = h_ref[...]
        if prebn:
            hh = _bn(hh)
        xw = _dot(hh, Wn_ref[...])
        u1 = _dot(hh, We1_ref[...])
        u2 = _dot(hh, We2_ref[...])
        s1 = _dot(xw, a1_ref[...])
        s2 = _dot(xw, a2_ref[...])
        z127 = jnp.zeros((N, 127), jnp.float32)
        src_ref[...] = jnp.concatenate([u1, xw, s1, z127], axis=1)
        dst_ref[...] = jnp.concatenate([u2, s2, z127], axis=1)
        xw_ref[...] = xw

    return _tc_call(
        body,
        [jax.ShapeDtypeStruct((N, 384), jnp.float32),
         jax.ShapeDtypeStruct((N, 256), jnp.float32),
         jax.ShapeDtypeStruct((N, 128), jnp.float32)],
        h, Wn, We1, We2, a1, a2)


def _edge_egat_feat(Gs, Gd, ea, We3, be, a3, ga, gb):
    def body(gs_ref, gd_ref, ea_ref, we3_ref, be_ref, a3_ref, ga_ref, gb_ref,
             v_ref, vd_ref, edge_ref, attn_ref):
        gs = gs_ref[...]
        gd = gd_ref[...]
        t = _dot(ea_ref[...], we3_ref[...])
        eh = jax.nn.relu(gs[:, :128] + gd[:, :128] + t + be_ref[...])
        lg = _leaky(gs[:, 256:257] + gd[:, 128:129] + _dot(eh, a3_ref[...]))
        e = jnp.exp(lg)
        v_ref[...] = e * (gs[:, 128:256] + eh)
        vd_ref[...] = jnp.concatenate(
            [e, jnp.zeros((BE, 127), jnp.float32)], axis=1)
        edge_ref[...] = jax.nn.relu(eh)
        attn_ref[...] = jax.nn.sigmoid(_dot(eh, ga_ref[...]) + gb_ref[...])

    grid = (NBLK,)
    eb = lambda w: pl.BlockSpec((BE, w), lambda i: (i, 0))
    fullb = lambda s: pl.BlockSpec(s, lambda i: (0, 0))
    return pl.pallas_call(
        body,
        grid=grid,
        in_specs=[eb(384), eb(256), eb(FE), fullb((FE, 128)), fullb((1, 128)),
                  fullb((128, 1)), fullb((128, 1)), fullb((1, 1))],
        out_specs=[eb(128), eb(128), eb(128), eb(1)],
        out_shape=[jax.ShapeDtypeStruct((EP, 128), jnp.float32),
                   jax.ShapeDtypeStruct((EP, 128), jnp.float32),
                   jax.ShapeDtypeStruct((EP, 128), jnp.float32),
                   jax.ShapeDtypeStruct((EP, 1), jnp.float32)],
    )(Gs, Gd, ea, We3, be, a3, ga, gb)


def _node_gat(am, ad, prev, b_or_xw_is_xw, W, asrcF, adstF, S4, R4):
    def body(am_ref, ad_ref, prev_ref, w_ref, asrc_ref, adst_ref, s4_ref,
             r4_ref, src_ref, dst_ref):
        num = am_ref[:N]
        d = ad_ref[:N]
        if b_or_xw_is_xw:
            h = prev_ref[...] + num / (d[:, 0:1] + 1e-16)
        else:
            den = _dot(d[:, 0:4], r4_ref[...]) + 1e-16
            h = num / den + prev_ref[...]
        h = _bn(jax.nn.relu(h))
        xwg = _dot(h, w_ref[...])
        asn = _dot(xwg * asrc_ref[...], s4_ref[...])
        adn = _dot(xwg * adst_ref[...], s4_ref[...])
        z124 = jnp.zeros((N, 124), jnp.float32)
        src_ref[...] = jnp.concatenate([xwg, asn, z124], axis=1)
        dst_ref[...] = jnp.concatenate([adn, z124], axis=1)

    return _tc_call(
        body,
        [jax.ShapeDtypeStruct((N, 256), jnp.float32),
         jax.ShapeDtypeStruct((N, 128), jnp.float32)],
        am, ad, prev, W, asrcF, adstF, S4, R4)


def _edge_gat(Gs, Gd, edge, We, aedgF, S4, R4):
    def body(gs_ref, gd_ref, edge_ref, we_ref, aedg_ref, s4_ref, r4_ref,
             v_ref, vd_ref):
        gs = gs_ref[...]
        ew = _dot(edge_ref[...], we_ref[...])
        ae = _dot(ew * aedg_ref[...], s4_ref[...])
        aa = _leaky(gs[:, 128:132] + gd_ref[...][:, :4] + ae)
        e = jnp.exp(aa)
        v_ref[...] = _dot(e, r4_ref[...]) * gs[:, :128]
        vd_ref[...] = jnp.concatenate(
            [e, jnp.zeros((BE, 124), jnp.float32)], axis=1)

    eb = lambda w: pl.BlockSpec((BE, w), lambda i: (i, 0))
    fullb = lambda s: pl.BlockSpec(s, lambda i: (0, 0))
    return pl.pallas_call(
        body,
        grid=(NBLK,),
        in_specs=[eb(256), eb(128), eb(128), fullb((128, 128)),
                  fullb((1, 128)), fullb((128, 4)), fullb((4, 128))],
        out_specs=[eb(128), eb(128)],
        out_shape=[jax.ShapeDtypeStruct((EP, 128), jnp.float32),
                   jax.ShapeDtypeStruct((EP, 128), jnp.float32)],
    )(Gs, Gd, edge, We, aedgF, S4, R4)


def _node_post(am, ad, b3, R4, Wna, bna):
    def body(am_ref, ad_ref, b3_ref, r4_ref, wna_ref, bna_ref,
             xcb_ref, xob_ref):
        num = am_ref[:N]
        d = ad_ref[:N]
        den = _dot(d[:, 0:4], r4_ref[...]) + 1e-16
        h = jax.nn.relu(num / den + b3_ref[...])
        lg = _dot(h, wna_ref[...]) + bna_ref[...]
        m = jnp.max(lg, axis=1, keepdims=True)
        ex = jnp.exp(lg - m)
        natt = ex / jnp.sum(ex, axis=1, keepdims=True)
        xcb_ref[...] = _bn(natt[:, 0:1] * h)
        xob_ref[...] = _bn(natt[:, 1:2] * h)

    return _tc_call(
        body,
        [jax.ShapeDtypeStruct((N, 128), jnp.float32),
         jax.ShapeDtypeStruct((N, 128), jnp.float32)],
        am, ad, b3, R4, Wna, bna)


def _edge_stats(edge, attn):
    def body(edge_ref, attn_ref, out_ref):
        pid = pl.program_id(0)
        rid = pid * BE + lax.broadcasted_iota(jnp.int32, (BE, 1), 0)
        mask = (rid < E).astype(jnp.float32)
        at = attn_ref[...]
        ed = edge_ref[...]
        ec = mask * (at * ed)
        eo = mask * ((1.0 - at) * ed)
        part = jnp.concatenate(
            [jnp.sum(ec, 0, keepdims=True), jnp.sum(ec * ec, 0, keepdims=True),
             jnp.sum(eo, 0, keepdims=True), jnp.sum(eo * eo, 0, keepdims=True),
             jnp.zeros((4, 128), jnp.float32)], axis=0)

        @pl.when(pid == 0)
        def _():
            out_ref[...] = jnp.zeros((8, 128), jnp.float32)

        out_ref[...] += part

    eb = lambda w: pl.BlockSpec((BE, w), lambda i: (i, 0))
    return pl.pallas_call(
        body,
        grid=(NBLK,),
        in_specs=[eb(128), eb(1)],
        out_specs=pl.BlockSpec((8, 128), lambda i: (0, 0)),
        out_shape=jax.ShapeDtypeStruct((8, 128), jnp.float32),
    )(edge, attn)


def _edge_egat2(Gs, Gd, edge, attn, stats, We3, be, a3, is_ctx):
    def body(gs_ref, gd_ref, edge_ref, attn_ref, st_ref, we3_ref, be_ref,
             a3_ref, v_ref, vd_ref):
        gs = gs_ref[...]
        gd = gd_ref[...]
        at = attn_ref[...]
        w = at if is_ctx else (1.0 - at)
        r = 0 if is_ctx else 2
        mu = st_ref[r:r + 1, :] * (1.0 / E)
        var = st_ref[r + 1:r + 2, :] * (1.0 / E) - mu * mu
        ea = (w * edge_ref[...] - mu) * lax.rsqrt(var + 1e-5) + 1e-4
        t = _dot(ea, we3_ref[...])
        eh = jax.nn.relu(gs[:, :128] + gd[:, :128] + t + be_ref[...])
        lg = _leaky(gs[:, 256:257] + gd[:, 128:129] + _dot(eh, a3_ref[...]))
        e = jnp.exp(lg)
        v_ref[...] = e * (gs[:, 128:256] + eh)
        vd_ref[...] = jnp.concatenate(
            [e, jnp.zeros((BE, 127), jnp.float32)], axis=1)

    eb = lambda w: pl.BlockSpec((BE, w), lambda i: (i, 0))
    fullb = lambda s: pl.BlockSpec(s, lambda i: (0, 0))
    return pl.pallas_call(
        body,
        grid=(NBLK,),
        in_specs=[eb(384), eb(256), eb(128), eb(1), fullb((8, 128)),
                  fullb((128, 128)), fullb((1, 128)), fullb((128, 1))],
        out_specs=[eb(128), eb(128)],
        out_shape=[jax.ShapeDtypeStruct((EP, 128), jnp.float32),
                   jax.ShapeDtypeStruct((EP, 128), jnp.float32)],
    )(Gs, Gd, edge, attn, stats, We3, be, a3)


def _pool_readout(am_c, ad_c, am_o, ad_o, xwc, xwo, batch2d, P,
                  w1c, b1c, w2c, b2c, w1o, b1o, w2o, b2o,
                  w1co, b1co, w2co, b2co):
    def body(amc_ref, adc_ref, amo_ref, ado_ref, xwc_ref, xwo_ref,
             b_ref, p_ref,
             w1c_ref, b1c_ref, w2c_ref, b2c_ref,
             w1o_ref, b1o_ref, w2o_ref, b2o_ref,
             w1co_ref, b1co_ref, w2co_ref, b2co_ref,
             oc_ref, oo_ref, oco_ref):
        def node_out(am_r, ad_r, xw_r):
            num = am_r[:N]
            d = ad_r[:N]
            return jax.nn.relu(xw_r[...] + num / (d[:, 0:1] + 1e-16))

        rc = node_out(amc_ref, adc_ref, xwc_ref)
        ro = node_out(amo_ref, ado_ref, xwo_ref)
        gids = lax.broadcasted_iota(jnp.int32, (1, G), 1)
        onehot = (b_ref[...] == gids).astype(jnp.float32)
        pc = lax.dot_general(onehot, rc, (((0,), (0,)), ((), ())),
                             preferred_element_type=jnp.float32)
        po = lax.dot_general(onehot, ro, (((0,), (0,)), ((), ())),
                             preferred_element_type=jnp.float32)

        def readout(z, w1_r, b1_r, w2_r, b2_r):
            z = _bn(z)
            z = jax.nn.relu(_dot(z, w1_r[...]) + b1_r[...])
            z = _bn(z)
            z = _dot(z, w2_r[...]) + b2_r[...]
            m = jnp.max(z, axis=1, keepdims=True)
            zz = z - m
            return zz - jnp.log(jnp.sum(jnp.exp(zz), axis=1, keepdims=True))

        oc_ref[...] = readout(pc, w1c_ref, b1c_ref, w2c_ref, b2c_ref)
        oo_ref[...] = readout(po, w1o_ref, b1o_ref, w2o_ref, b2o_ref)
        pcp = _dot(p_ref[...], pc)
        oco_ref[...] = readout(pcp + po, w1co_ref, b1co_ref, w2co_ref,
                               b2co_ref)

    return _tc_call(
        body,
        [jax.ShapeDtypeStruct((G, C), jnp.float32)] * 3,
        am_c, ad_c, am_o, ad_o, xwc, xwo, batch2d, P,
        w1c, b1c, w2c, b2c, w1o, b1o, w2o, b2o, w1co, b1co, w2co, b2co)


_S4_np = np.zeros((128, 4), np.float32)
for _h in range(4):
    _S4_np[32 * _h:32 * (_h + 1), _h] = 1.0
_R4_np = _S4_np.T.copy()


def kernel(x, edge_index, edge_attr, batch, params):
    S4 = jnp.asarray(_S4_np)
    R4 = jnp.asarray(_R4_np)
    perm = jax.random.permutation(jax.random.key(42), G)
    P = (perm[:, None] == jnp.arange(G)[None, :]).astype(jnp.float32)

    row = jnp.concatenate(
        [edge_index[0].astype(jnp.int32), jnp.zeros((EP - E,), jnp.int32)])
    col = jnp.concatenate(
        [edge_index[1].astype(jnp.int32), jnp.zeros((EP - E,), jnp.int32)])
    col_s = jnp.concatenate(
        [edge_index[1].astype(jnp.int32), jnp.full((EP - E,), N, jnp.int32)])
    ea_p = jnp.concatenate(
        [edge_attr, jnp.zeros((EP - E, FE), jnp.float32)], axis=0)

    def split_egat(p):
        We = p['We']
        a = p['a']
        return dict(
            Wn=p['Wn'], We1=We[:128], We2=We[128:256], We3=We[256:],
            be=p['be'].reshape(1, 128), a1=a[:128, None], a2=a[128:256, None],
            a3=a[256:384, None], ga=p['ga'][:, None],
            gb=p['gb'].reshape(1, 1))

    tok0 = jnp.zeros((8, 128), jnp.float32)

    pf = split_egat(params['feat'])
    Tsrc, Tdst, xw0 = _node_egat(
        x, pf['Wn'], pf['We1'], pf['We2'], pf['a1'], pf['a2'], prebn=True)
    Gs, Gd = _sc_gather2(Tsrc, Tdst, row, col, tok0)
    V, Vd, edge, attn = _edge_egat_feat(
        Gs, Gd, ea_p, pf['We3'], pf['be'], pf['a3'], pf['ga'], pf['gb'])
    am, ad = _sc_scatter_add2(V, Vd, col_s)

    prev = xw0
    first = True
    for p in params['convs']:
        Tsrc, Tdst = _node_gat(
            am, ad, prev, first, p['W'], p['asrc'].reshape(1, 128),
            p['adst'].reshape(1, 128), S4, R4)
        Gs, Gd = _sc_gather2(Tsrc, Tdst, row, col, tok0)
        V, Vd = _edge_gat(Gs, Gd, edge, p['We'], p['aedge'].reshape(1, 128),
                          S4, R4)
        am, ad = _sc_scatter_add2(V, Vd, col_s)
        prev = p['b'].reshape(1, 128)
        first = False

    xcb, xob = _node_post(
        am, ad, params['convs'][2]['b'].reshape(1, 128), R4,
        params['natt']['W'], params['natt']['b'].reshape(1, 2))

    stats = _edge_stats(edge, attn)

    def branch(h, pr, is_ctx, tok):
        pp = split_egat(pr)
        Ts, Td, xwb = _node_egat(
            h, pp['Wn'], pp['We1'], pp['We2'], pp['a1'], pp['a2'],
            prebn=False)
        Gsb, Gdb = _sc_gather2(Ts, Td, row, col, tok)
        Vb, Vdb = _edge_egat2(Gsb, Gdb, edge, attn, stats, pp['We3'],
                              pp['be'], pp['a3'], is_ctx)
        amb, adb = _sc_scatter_add2(Vb, Vdb, col_s)
        return amb, adb, xwb

    am_c, ad_c, xwc = branch(xcb, params['ctx'], True, tok0)
    tok1 = lax.slice(am_c, (0, 0), (8, 128))
    am_o, ad_o, xwo = branch(xob, params['obj'], False, tok1)

    b2d = batch.astype(jnp.int32)[:, None]
    fc = params['fc1_c'], params['fc2_c']
    fo = params['fc1_o'], params['fc2_o']
    fco = params['fc1_co'], params['fc2_co']
    oc, oo, oco = _pool_readout(
        am_c, ad_c, am_o, ad_o, xwc, xwo, b2d, P,
        fc[0]['W'], fc[0]['b'].reshape(1, H), fc[1]['W'],
        fc[1]['b'].reshape(1, C),
        fo[0]['W'], fo[0]['b'].reshape(1, H), fo[1]['W'],
        fo[1]['b'].reshape(1, C),
        fco[0]['W'], fco[0]['b'].reshape(1, H), fco[1]['W'],
        fco[1]['b'].reshape(1, C))
    return oc, oo, oco
